# baseline (speedup 1.0000x reference)
import functools
import math

import jax
import jax.numpy as jnp
from jax import lax
from jax.experimental import pallas as pl
from jax.experimental.pallas import tpu as pltpu
from jax.experimental.pallas import tpu_sc as plsc

VOCAB = 100000
D = 128
SCALE = math.sqrt(D)

NC = 2
NS = 16
NW = NC * NS

CH = 128
NBUF = 6
LEAD = 3


def _make_gather(B):
    assert B % (8 * NW) == 0
    bpw = B // NW
    assert bpw % CH == 0
    nch = bpw // CH
    assert nch >= 2 * NBUF and NBUF == 2 * LEAD
    mesh = plsc.VectorSubcoreMesh(core_axis_name="c", subcore_axis_name="s")

    @functools.partial(
        pl.kernel,
        mesh=mesh,
        out_type=jax.ShapeDtypeStruct((B, D), jnp.float32),
        scratch_types=[
            pltpu.VMEM((bpw,), jnp.int32),
            tuple(pltpu.VMEM((CH, D), jnp.float32) for _ in range(NBUF)),
            tuple(pltpu.SemaphoreType.DMA for _ in range(NBUF)),
            tuple(pltpu.SemaphoreType.DMA for _ in range(NBUF)),
        ],
    )
    def gather_k(table_hbm, idx_hbm, out_hbm, idx_v, bufs, gsems, ssems):
        wid = lax.axis_index("s") * NC + lax.axis_index("c")
        base = wid * bpw
        pltpu.sync_copy(idx_hbm.at[pl.ds(base, bpw)], idx_v)

        def start_gather(c, b):
            pltpu.async_copy(
                table_hbm.at[idx_v.at[pl.ds(c * CH, CH)]], bufs[b], gsems[b]
            )

        def wait_gather(c, b):
            pltpu.make_async_copy(
                table_hbm.at[idx_v.at[pl.ds(c * CH, CH)]], bufs[b], gsems[b]
            ).wait()

        def scale_buf(b):
            buf = bufs[b]

            def row(r, carry):
                for j in range(D // 16):
                    sl = pl.ds(j * 16, 16)
                    buf[r, sl] = buf[r, sl] * SCALE
                return carry

            lax.fori_loop(0, CH, row, 0, unroll=2)

        def start_store(c, b):
            pltpu.async_copy(
                bufs[b], out_hbm.at[pl.ds(base + c * CH, CH)], ssems[b]
            )

        def wait_store(c, b):
            pltpu.make_async_copy(
                bufs[b], out_hbm.at[pl.ds(base + c * CH, CH)], ssems[b]
            ).wait()

        for c in range(2 * LEAD):
            start_gather(c, c % NBUF)
        for c in range(LEAD):
            wait_gather(c, c % NBUF)
            scale_buf(c % NBUF)
            start_store(c, c % NBUF)

        def group(g, carry):
            c0 = LEAD + g * NBUF
            for b in range(NBUF):
                c = c0 + b
                cur = (LEAD + b) % NBUF
                bb = (LEAD + b + LEAD) % NBUF
                wait_store(c - LEAD, bb)
                start_gather(c + LEAD, bb)
                wait_gather(c, cur)
                scale_buf(cur)
                start_store(c, cur)
            return carry

        ngroups = (nch - 2 * LEAD) // NBUF
        lax.fori_loop(0, ngroups, group, 0, unroll=False)

        rem = LEAD + ngroups * NBUF
        for c in range(rem, nch - LEAD):
            wait_store(c - LEAD, (c + LEAD) % NBUF)
            start_gather(c + LEAD, (c + LEAD) % NBUF)
            wait_gather(c, c % NBUF)
            scale_buf(c % NBUF)
            start_store(c, c % NBUF)

        for c in range(max(nch - LEAD, LEAD), nch):
            wait_gather(c, c % NBUF)
            scale_buf(c % NBUF)
            start_store(c, c % NBUF)
        for c in range(nch - NBUF, nch):
            wait_store(c, c % NBUF)

    return gather_k


@jax.jit
def kernel(x, table):
    flat_idx = x.reshape(-1).astype(jnp.int32)
    B = flat_idx.shape[0]
    out = _make_gather(B)(table, flat_idx)
    return out.reshape(x.shape + (D,))
